# baseline (speedup 1.0000x reference)
import functools

import jax
import jax.numpy as jnp
from jax import lax
from jax.experimental import pallas as pl
from jax.experimental.pallas import tpu as pltpu
from jax.experimental.pallas import tpu_sc as plsc

EMBEDDING_DIM = 32
CHUNK = 640
NBUF = 4
KSUB = 4
SUB = CHUNK // KSUB


@jax.jit
def _embedding_lookup(idx_flat, table):
    info = plsc.get_sparse_core_info()
    num_workers = info.num_cores * info.num_subcores
    b_total = idx_flat.shape[0]
    b_per_w = b_total // num_workers
    n_chunks = b_per_w // CHUNK

    mesh = plsc.VectorSubcoreMesh(core_axis_name="c", subcore_axis_name="s")

    @functools.partial(
        pl.kernel,
        mesh=mesh,
        out_type=jax.ShapeDtypeStruct((b_total, EMBEDDING_DIM), jnp.float32),
        scratch_types=[
            pltpu.VMEM((b_per_w,), jnp.int32),
            pltpu.VMEM((NBUF, CHUNK, EMBEDDING_DIM), jnp.float32),
        ]
        + [pltpu.SemaphoreType.DMA] * (2 * NBUF),
        compiler_params=pltpu.CompilerParams(use_tc_tiling_on_sc=False),
    )
    def emb_kernel(idx_hbm, table_hbm, out_hbm, idx_v, rows_v, *sems):
        gsem = sems[:NBUF]
        ssem = sems[NBUF:]
        wid = lax.axis_index("s") * info.num_cores + lax.axis_index("c")
        base = wid * b_per_w

        def gather_start(g, b):
            for k in range(KSUB):
                pltpu.async_copy(
                    table_hbm.at[idx_v.at[pl.ds(g * CHUNK + k * SUB, SUB)]],
                    rows_v.at[b].at[pl.ds(k * SUB, SUB)],
                    gsem[b],
                )

        def gather_wait(g, b):
            for k in range(KSUB):
                pltpu.make_async_copy(
                    table_hbm.at[idx_v.at[pl.ds(g * CHUNK + k * SUB, SUB)]],
                    rows_v.at[b].at[pl.ds(k * SUB, SUB)],
                    gsem[b],
                ).wait()

        def store_start(g, b):
            pltpu.async_copy(
                rows_v.at[b], out_hbm.at[pl.ds(base + g * CHUNK, CHUNK)], ssem[b]
            )

        def store_wait(g, b):
            pltpu.make_async_copy(
                rows_v.at[b], out_hbm.at[pl.ds(base + g * CHUNK, CHUNK)], ssem[b]
            ).wait()

        pltpu.sync_copy(idx_hbm.at[pl.ds(base, b_per_w)], idx_v)

        gather_start(0, 0)
        gather_start(1, 1)

        def chunk_step(g, issue_ahead, wait_prev_store):
            b = g % NBUF
            gather_wait(g, b)
            store_start(g, b)
            if issue_ahead:
                b2 = (g + 2) % NBUF
                if wait_prev_store:
                    store_wait(g - 2, b2)
                gather_start(g + 2, b2)

        chunk_step(0, True, False)
        chunk_step(1, True, False)

        @pl.loop(2, n_chunks - 2, step=NBUF)
        def _steady(outer):
            for j in range(NBUF):
                g = outer + j
                b = (2 + j) % NBUF
                gather_wait(g, b)
                store_start(g, b)
                b2 = (2 + j + 2) % NBUF
                store_wait(g - 2, b2)
                gather_start(g + 2, b2)

        for gg in (n_chunks - 2, n_chunks - 1):
            b = gg % NBUF
            gather_wait(gg, b)
            store_start(gg, b)

        for gg in range(n_chunks - NBUF, n_chunks):
            store_wait(gg, gg % NBUF)

    return emb_kernel(idx_flat, table)


def kernel(x, table):
    idx_flat = x.reshape(-1).astype(jnp.int32)
    out = _embedding_lookup(idx_flat, table)
    return out.reshape(x.shape + (EMBEDDING_DIM,))
